# baseline (speedup 1.0000x reference)
import functools

import jax
import jax.numpy as jnp
from jax import lax
from jax.experimental import pallas as pl
from jax.experimental.pallas import tpu as pltpu
from jax.experimental.pallas import tpu_sc as plsc

_NC = 2
_NS = 16
_L = 16
_NW = _NC * _NS

_B = 16384
_D = 32
_BPW = _B // _NW


def _mf_body(u_hbm, v_hbm, ue_hbm, ve_hbm, out_hbm,
             ui_v, vi_v, ue_v, ve_v, o_v, sem):
    wid = lax.axis_index("s") * _NC + lax.axis_index("c")
    base = wid * _BPW

    pltpu.sync_copy(u_hbm.at[pl.ds(base, _BPW)], ui_v)
    pltpu.sync_copy(v_hbm.at[pl.ds(base, _BPW)], vi_v)

    cu = pltpu.async_copy(ue_hbm.at[ui_v], ue_v, sem)
    cv = pltpu.async_copy(ve_hbm.at[vi_v], ve_v, sem)
    cu.wait()
    cv.wait()

    iota = lax.iota(jnp.int32, _L)

    def group_body(g, _):
        row0 = g * _L
        rows = row0 + iota

        def d_body(d, acc):
            cols = jnp.zeros((_L,), jnp.int32) + d
            a = plsc.load_gather(ue_v, [rows, cols])
            b = plsc.load_gather(ve_v, [rows, cols])
            return acc + a * b

        acc = lax.fori_loop(0, _D, d_body, jnp.zeros((_L,), jnp.float32))
        o_v[pl.ds(row0, _L)] = 4.0 / (1.0 + jnp.exp(-acc)) + 1.0
        return 0

    lax.fori_loop(0, _BPW // _L, group_body, 0)

    pltpu.sync_copy(o_v, out_hbm.at[pl.ds(base, _BPW)])


def kernel(u, v, user_emb, movie_emb):
    mesh = plsc.VectorSubcoreMesh(core_axis_name="c", subcore_axis_name="s")
    run = functools.partial(
        pl.kernel,
        out_type=jax.ShapeDtypeStruct((_B,), jnp.float32),
        mesh=mesh,
        compiler_params=pltpu.CompilerParams(
            needs_layout_passes=False, use_tc_tiling_on_sc=False
        ),
        scratch_types=[
            pltpu.VMEM((_BPW,), jnp.int32),
            pltpu.VMEM((_BPW,), jnp.int32),
            pltpu.VMEM((_BPW, _D), jnp.float32),
            pltpu.VMEM((_BPW, _D), jnp.float32),
            pltpu.VMEM((_BPW,), jnp.float32),
            pltpu.SemaphoreType.DMA,
        ],
    )(_mf_body)
    return run(u, v, user_emb, movie_emb)
